# baseline (speedup 1.0000x reference)
import jax
import jax.numpy as jnp
from jax.experimental import pallas as pl
from jax.experimental.pallas import tpu as pltpu
from jax.experimental.pallas import tpu_sc as plsc


_GATHER_WINDOW = 128


def _sc_gather(user_emb, movie_emb, uids, mids):
    B = uids.shape[0]
    D = user_emb.shape[1]
    mesh = plsc.VectorSubcoreMesh(core_axis_name="core", subcore_axis_name="subcore")

    uids2 = uids.reshape(1, B)
    mids2 = mids.reshape(1, B)

    @pl.kernel(
        out_type=(
            jax.ShapeDtypeStruct((B, D), user_emb.dtype),
            jax.ShapeDtypeStruct((B, D), movie_emb.dtype),
        ),
        mesh=mesh,
    )
    def gather_kernel(ue_hbm, me_hbm, ui_hbm, mi_hbm, ou_hbm, om_hbm):
        def body(ui_vmem, mi_vmem, ou_vmem, om_vmem):
            pltpu.sync_copy(ue_hbm.at[ui_vmem.at[0]], ou_vmem)
            pltpu.sync_copy(me_hbm.at[mi_vmem.at[0]], om_vmem)

        pltpu.emit_pipeline(
            body,
            grid=(B // _GATHER_WINDOW,),
            in_specs=[
                pl.BlockSpec((1, _GATHER_WINDOW), lambda i: (0, i)),
                pl.BlockSpec((1, _GATHER_WINDOW), lambda i: (0, i)),
            ],
            out_specs=[
                pl.BlockSpec((_GATHER_WINDOW, D), lambda i: (i, 0)),
                pl.BlockSpec((_GATHER_WINDOW, D), lambda i: (i, 0)),
            ],
            core_axis_name=("core", "subcore"),
            dimension_semantics=(pltpu.PARALLEL,),
        )(ui_hbm, mi_hbm, ou_hbm, om_hbm)

    return gather_kernel(user_emb, movie_emb, uids2, mids2)


_MLP_BLOCK = 2048


def _mlp_body(u_ref, m_ref, w1u_ref, w1m_ref, b1_ref, w2_ref, b2_ref,
              w3_ref, b3_ref, o_ref):
    h = jnp.dot(u_ref[...], w1u_ref[...], preferred_element_type=jnp.float32)
    h = h + jnp.dot(m_ref[...], w1m_ref[...], preferred_element_type=jnp.float32)
    h = jnp.maximum(h + b1_ref[...], 0.0)
    h2t = jax.lax.dot_general(w2_ref[...], h, (((1,), (1,)), ((), ())),
                              preferred_element_type=jnp.float32)
    h2t = jnp.maximum(h2t + b2_ref[...], 0.0)
    ot = jax.lax.dot_general(w3_ref[...], h2t, (((1,), (0,)), ((), ())),
                             preferred_element_type=jnp.float32)
    o_ref[...] = ot + b3_ref[0, 0]


def _mlp(u, m, W1, b1, W2, b2, W3, b3):
    B, D = u.shape
    w1u_t = W1[:, :D].T
    w1m_t = W1[:, D:].T
    b1_2d = b1.reshape(1, -1)
    b2_2d = b2.reshape(-1, 1)
    w3_2d = W3
    b3_2d = b3.reshape(1, 1)

    grid = (B // _MLP_BLOCK,)
    out_t = pl.pallas_call(
        _mlp_body,
        grid=grid,
        in_specs=[
            pl.BlockSpec((_MLP_BLOCK, D), lambda i: (i, 0)),
            pl.BlockSpec((_MLP_BLOCK, D), lambda i: (i, 0)),
            pl.BlockSpec(w1u_t.shape, lambda i: (0, 0)),
            pl.BlockSpec(w1m_t.shape, lambda i: (0, 0)),
            pl.BlockSpec(b1_2d.shape, lambda i: (0, 0)),
            pl.BlockSpec(W2.shape, lambda i: (0, 0)),
            pl.BlockSpec(b2_2d.shape, lambda i: (0, 0)),
            pl.BlockSpec(w3_2d.shape, lambda i: (0, 0)),
            pl.BlockSpec(b3_2d.shape, lambda i: (0, 0)),
        ],
        out_specs=pl.BlockSpec((1, _MLP_BLOCK), lambda i: (0, i)),
        out_shape=jax.ShapeDtypeStruct((1, B), jnp.float32),
    )(u, m, w1u_t, w1m_t, b1_2d, W2, b2_2d, w3_2d, b3_2d)
    return out_t.reshape(B)


_N_CHUNKS = 2


def kernel(user_ids, movie_ids, user_emb, movie_emb, W1, b1, W2, b2, W3, b3):
    B = user_ids.shape[0]
    Bc = B // _N_CHUNKS
    outs = []
    for c in range(_N_CHUNKS):
        u, m = _sc_gather(
            user_emb, movie_emb,
            jax.lax.dynamic_slice_in_dim(user_ids, c * Bc, Bc),
            jax.lax.dynamic_slice_in_dim(movie_ids, c * Bc, Bc),
        )
        outs.append(_mlp(u, m, W1, b1, W2, b2, W3, b3))
    return jnp.concatenate(outs)


# speedup vs baseline: 1.0646x; 1.0646x over previous
import jax
import jax.numpy as jnp
from jax.experimental import pallas as pl
from jax.experimental.pallas import tpu as pltpu
from jax.experimental.pallas import tpu_sc as plsc


_GATHER_WINDOW = 128


def _sc_gather(user_emb, movie_emb, uids, mids):
    B = uids.shape[0]
    D = user_emb.shape[1]
    mesh = plsc.VectorSubcoreMesh(core_axis_name="core", subcore_axis_name="subcore")

    uids2 = uids.reshape(1, B)
    mids2 = mids.reshape(1, B)

    @pl.kernel(
        out_type=(
            jax.ShapeDtypeStruct((B, D), user_emb.dtype),
            jax.ShapeDtypeStruct((B, D), movie_emb.dtype),
        ),
        mesh=mesh,
    )
    def gather_kernel(ue_hbm, me_hbm, ui_hbm, mi_hbm, ou_hbm, om_hbm):
        def body(ui_vmem, mi_vmem, ou_vmem, om_vmem):
            pltpu.sync_copy(ue_hbm.at[ui_vmem.at[0]], ou_vmem)
            pltpu.sync_copy(me_hbm.at[mi_vmem.at[0]], om_vmem)

        pltpu.emit_pipeline(
            body,
            grid=(B // _GATHER_WINDOW,),
            in_specs=[
                pl.BlockSpec((1, _GATHER_WINDOW), lambda i: (0, i)),
                pl.BlockSpec((1, _GATHER_WINDOW), lambda i: (0, i)),
            ],
            out_specs=[
                pl.BlockSpec((_GATHER_WINDOW, D), lambda i: (i, 0)),
                pl.BlockSpec((_GATHER_WINDOW, D), lambda i: (i, 0)),
            ],
            core_axis_name=("core", "subcore"),
            dimension_semantics=(pltpu.PARALLEL,),
        )(ui_hbm, mi_hbm, ou_hbm, om_hbm)

    return gather_kernel(user_emb, movie_emb, uids2, mids2)


_MLP_BLOCK = 2048


def _mlp_body(u_ref, m_ref, w1u_ref, w1m_ref, b1_ref, w2_ref, b2_ref,
              w3_ref, b3_ref, o_ref):
    h = jnp.dot(u_ref[...], w1u_ref[...], preferred_element_type=jnp.float32)
    h = h + jnp.dot(m_ref[...], w1m_ref[...], preferred_element_type=jnp.float32)
    h = jnp.maximum(h + b1_ref[...], 0.0)
    h2t = jax.lax.dot_general(w2_ref[...], h, (((1,), (1,)), ((), ())),
                              preferred_element_type=jnp.float32)
    h2t = jnp.maximum(h2t + b2_ref[...], 0.0)
    ot = jax.lax.dot_general(w3_ref[...], h2t, (((1,), (0,)), ((), ())),
                             preferred_element_type=jnp.float32)
    o_ref[...] = ot + b3_ref[0, 0]


def _mlp(u, m, W1, b1, W2, b2, W3, b3):
    B, D = u.shape
    w1u_t = W1[:, :D].T
    w1m_t = W1[:, D:].T
    b1_2d = b1.reshape(1, -1)
    b2_2d = b2.reshape(-1, 1)
    w3_2d = W3
    b3_2d = b3.reshape(1, 1)

    grid = (B // _MLP_BLOCK,)
    out_t = pl.pallas_call(
        _mlp_body,
        grid=grid,
        in_specs=[
            pl.BlockSpec((_MLP_BLOCK, D), lambda i: (i, 0)),
            pl.BlockSpec((_MLP_BLOCK, D), lambda i: (i, 0)),
            pl.BlockSpec(w1u_t.shape, lambda i: (0, 0)),
            pl.BlockSpec(w1m_t.shape, lambda i: (0, 0)),
            pl.BlockSpec(b1_2d.shape, lambda i: (0, 0)),
            pl.BlockSpec(W2.shape, lambda i: (0, 0)),
            pl.BlockSpec(b2_2d.shape, lambda i: (0, 0)),
            pl.BlockSpec(w3_2d.shape, lambda i: (0, 0)),
            pl.BlockSpec(b3_2d.shape, lambda i: (0, 0)),
        ],
        out_specs=pl.BlockSpec((1, _MLP_BLOCK), lambda i: (0, i)),
        out_shape=jax.ShapeDtypeStruct((1, B), jnp.float32),
    )(u, m, w1u_t, w1m_t, b1_2d, W2, b2_2d, w3_2d, b3_2d)
    return out_t.reshape(B)


def kernel(user_ids, movie_ids, user_emb, movie_emb, W1, b1, W2, b2, W3, b3):
    u, m = _sc_gather(user_emb, movie_emb, user_ids, movie_ids)
    return _mlp(u, m, W1, b1, W2, b2, W3, b3)


# speedup vs baseline: 1.0661x; 1.0015x over previous
import jax
import jax.numpy as jnp
from jax.experimental import pallas as pl
from jax.experimental.pallas import tpu as pltpu
from jax.experimental.pallas import tpu_sc as plsc


_GATHER_WINDOW = 128


def _sc_gather(user_emb, movie_emb, uids, mids):
    B = uids.shape[0]
    D = user_emb.shape[1]
    mesh = plsc.VectorSubcoreMesh(core_axis_name="core", subcore_axis_name="subcore")

    uids2 = uids.reshape(1, B)
    mids2 = mids.reshape(1, B)

    @pl.kernel(
        out_type=(
            jax.ShapeDtypeStruct((B, D), user_emb.dtype),
            jax.ShapeDtypeStruct((B, D), movie_emb.dtype),
        ),
        mesh=mesh,
    )
    def gather_kernel(ue_hbm, me_hbm, ui_hbm, mi_hbm, ou_hbm, om_hbm):
        def body(ui_vmem, mi_vmem, ou_vmem, om_vmem):
            pltpu.sync_copy(ue_hbm.at[ui_vmem.at[0]], ou_vmem)
            pltpu.sync_copy(me_hbm.at[mi_vmem.at[0]], om_vmem)

        pltpu.emit_pipeline(
            body,
            grid=(B // _GATHER_WINDOW,),
            in_specs=[
                pl.BlockSpec((1, _GATHER_WINDOW), lambda i: (0, i)),
                pl.BlockSpec((1, _GATHER_WINDOW), lambda i: (0, i)),
            ],
            out_specs=[
                pl.BlockSpec((_GATHER_WINDOW, D), lambda i: (i, 0)),
                pl.BlockSpec((_GATHER_WINDOW, D), lambda i: (i, 0)),
            ],
            core_axis_name=("core", "subcore"),
            dimension_semantics=(pltpu.PARALLEL,),
        )(ui_hbm, mi_hbm, ou_hbm, om_hbm)

    return gather_kernel(user_emb, movie_emb, uids2, mids2)


_MLP_BLOCK = 2048


def _mlp_body(u_ref, m_ref, w1_ref, b1_ref, w2_ref, b2_ref,
              w3_ref, b3_ref, o_ref):
    D = u_ref.shape[1]
    h = jax.lax.dot_general(u_ref[...], w1_ref[:, :D], (((1,), (1,)), ((), ())),
                            preferred_element_type=jnp.float32)
    h = h + jax.lax.dot_general(m_ref[...], w1_ref[:, D:], (((1,), (1,)), ((), ())),
                                preferred_element_type=jnp.float32)
    h = jnp.maximum(h + b1_ref[...], 0.0)
    h2t = jax.lax.dot_general(w2_ref[...], h, (((1,), (1,)), ((), ())),
                              preferred_element_type=jnp.float32)
    h2t = jnp.maximum(h2t + b2_ref[...], 0.0)
    ot = jax.lax.dot_general(w3_ref[...], h2t, (((1,), (0,)), ((), ())),
                             preferred_element_type=jnp.float32)
    o_ref[...] = ot + b3_ref[0, 0]


def _mlp(u, m, W1, b1, W2, b2, W3, b3):
    B, D = u.shape
    b1_2d = b1.reshape(1, -1)
    b2_2d = b2.reshape(-1, 1)
    b3_2d = b3.reshape(1, 1)

    grid = (B // _MLP_BLOCK,)
    out_t = pl.pallas_call(
        _mlp_body,
        grid=grid,
        in_specs=[
            pl.BlockSpec((_MLP_BLOCK, D), lambda i: (i, 0)),
            pl.BlockSpec((_MLP_BLOCK, D), lambda i: (i, 0)),
            pl.BlockSpec(W1.shape, lambda i: (0, 0)),
            pl.BlockSpec(b1_2d.shape, lambda i: (0, 0)),
            pl.BlockSpec(W2.shape, lambda i: (0, 0)),
            pl.BlockSpec(b2_2d.shape, lambda i: (0, 0)),
            pl.BlockSpec(W3.shape, lambda i: (0, 0)),
            pl.BlockSpec(b3_2d.shape, lambda i: (0, 0)),
        ],
        out_specs=pl.BlockSpec((1, _MLP_BLOCK), lambda i: (0, i)),
        out_shape=jax.ShapeDtypeStruct((1, B), jnp.float32),
        compiler_params=pltpu.CompilerParams(
            dimension_semantics=("parallel",),
        ),
    )(u, m, W1, b1_2d, W2, b2_2d, W3, b3_2d)
    return out_t.reshape(B)


def kernel(user_ids, movie_ids, user_emb, movie_emb, W1, b1, W2, b2, W3, b3):
    u, m = _sc_gather(user_emb, movie_emb, user_ids, movie_ids)
    return _mlp(u, m, W1, b1, W2, b2, W3, b3)


# speedup vs baseline: 1.1459x; 1.0748x over previous
import jax
import jax.numpy as jnp
from jax.experimental import pallas as pl
from jax.experimental.pallas import tpu as pltpu
from jax.experimental.pallas import tpu_sc as plsc


_GATHER_WINDOW = 128


def _sc_gather(user_emb, movie_emb, uids, mids):
    B = uids.shape[0]
    D = user_emb.shape[1]
    mesh = plsc.VectorSubcoreMesh(core_axis_name="core", subcore_axis_name="subcore")

    uids2 = uids.reshape(1, B)
    mids2 = mids.reshape(1, B)

    @pl.kernel(
        out_type=(
            jax.ShapeDtypeStruct((B, D), user_emb.dtype),
            jax.ShapeDtypeStruct((B, D), movie_emb.dtype),
        ),
        mesh=mesh,
    )
    def gather_kernel(ue_hbm, me_hbm, ui_hbm, mi_hbm, ou_hbm, om_hbm):
        def body(ui_vmem, mi_vmem, ou_vmem, om_vmem):
            pltpu.sync_copy(ue_hbm.at[ui_vmem.at[0]], ou_vmem)
            pltpu.sync_copy(me_hbm.at[mi_vmem.at[0]], om_vmem)

        pltpu.emit_pipeline(
            body,
            grid=(B // _GATHER_WINDOW,),
            in_specs=[
                pl.BlockSpec((1, _GATHER_WINDOW), lambda i: (0, i)),
                pl.BlockSpec((1, _GATHER_WINDOW), lambda i: (0, i)),
            ],
            out_specs=[
                pl.BlockSpec((_GATHER_WINDOW, D), lambda i: (i, 0)),
                pl.BlockSpec((_GATHER_WINDOW, D), lambda i: (i, 0)),
            ],
            core_axis_name=("core", "subcore"),
            dimension_semantics=(pltpu.PARALLEL,),
        )(ui_hbm, mi_hbm, ou_hbm, om_hbm)

    return gather_kernel(user_emb, movie_emb, uids2, mids2)


_MLP_BLOCK = 4096


def _mlp_body(u_ref, m_ref, w1_ref, b1_ref, w2_ref, b2_ref,
              w3_ref, b3_ref, o_ref):
    D = u_ref.shape[1]
    h = jax.lax.dot_general(u_ref[...], w1_ref[:, :D], (((1,), (1,)), ((), ())),
                            preferred_element_type=jnp.float32)
    h = h + jax.lax.dot_general(m_ref[...], w1_ref[:, D:], (((1,), (1,)), ((), ())),
                                preferred_element_type=jnp.float32)
    h = jnp.maximum(h + b1_ref[...], 0.0)
    h2t = jax.lax.dot_general(w2_ref[...], h, (((1,), (1,)), ((), ())),
                              preferred_element_type=jnp.float32)
    h2t = jnp.maximum(h2t + b2_ref[...], 0.0)
    ot = jax.lax.dot_general(w3_ref[...], h2t, (((1,), (0,)), ((), ())),
                             preferred_element_type=jnp.float32)
    o_ref[...] = ot + b3_ref[0, 0]


def _mlp(u, m, W1, b1, W2, b2, W3, b3):
    B, D = u.shape
    b1_2d = b1.reshape(1, -1)
    b2_2d = b2.reshape(-1, 1)
    b3_2d = b3.reshape(1, 1)

    grid = (B // _MLP_BLOCK,)
    out_t = pl.pallas_call(
        _mlp_body,
        grid=grid,
        in_specs=[
            pl.BlockSpec((_MLP_BLOCK, D), lambda i: (i, 0)),
            pl.BlockSpec((_MLP_BLOCK, D), lambda i: (i, 0)),
            pl.BlockSpec(W1.shape, lambda i: (0, 0)),
            pl.BlockSpec(b1_2d.shape, lambda i: (0, 0)),
            pl.BlockSpec(W2.shape, lambda i: (0, 0)),
            pl.BlockSpec(b2_2d.shape, lambda i: (0, 0)),
            pl.BlockSpec(W3.shape, lambda i: (0, 0)),
            pl.BlockSpec(b3_2d.shape, lambda i: (0, 0)),
        ],
        out_specs=pl.BlockSpec((1, _MLP_BLOCK), lambda i: (0, i)),
        out_shape=jax.ShapeDtypeStruct((1, B), jnp.float32),
        compiler_params=pltpu.CompilerParams(
            dimension_semantics=("parallel",),
        ),
    )(u, m, W1, b1_2d, W2, b2_2d, W3, b3_2d)
    return out_t.reshape(B)


def kernel(user_ids, movie_ids, user_emb, movie_emb, W1, b1, W2, b2, W3, b3):
    u, m = _sc_gather(user_emb, movie_emb, user_ids, movie_ids)
    return _mlp(u, m, W1, b1, W2, b2, W3, b3)


# speedup vs baseline: 1.2092x; 1.0552x over previous
import jax
import jax.numpy as jnp
from jax.experimental import pallas as pl
from jax.experimental.pallas import tpu as pltpu
from jax.experimental.pallas import tpu_sc as plsc


_NUM_WORKERS = 32


def _sc_gather(user_emb, movie_emb, uids, mids):
    B = uids.shape[0]
    D = user_emb.shape[1]
    per_w = B // _NUM_WORKERS
    half = per_w // 2
    mesh = plsc.VectorSubcoreMesh(core_axis_name="core", subcore_axis_name="subcore")

    @pl.kernel(
        out_type=(
            jax.ShapeDtypeStruct((B, D), user_emb.dtype),
            jax.ShapeDtypeStruct((B, D), movie_emb.dtype),
        ),
        mesh=mesh,
        scratch_types=[
            pltpu.VMEM((half,), jnp.int32),
            pltpu.VMEM((half,), jnp.int32),
            pltpu.VMEM((half,), jnp.int32),
            pltpu.VMEM((half,), jnp.int32),
            pltpu.VMEM((half, D), jnp.float32),
            pltpu.VMEM((half, D), jnp.float32),
            pltpu.SemaphoreType.DMA,
            pltpu.SemaphoreType.DMA,
            pltpu.SemaphoreType.DMA,
        ],
    )
    def gather_kernel(ue_hbm, me_hbm, ui_hbm, mi_hbm, ou_hbm, om_hbm,
                      iu0, iu1, im0, im1, buf0, buf1, sem0, sem1, sem_idx):
        wid = (jax.lax.axis_index("subcore") * 2 + jax.lax.axis_index("core"))
        base = wid * per_w
        idx_cps = [
            pltpu.async_copy(ui_hbm.at[pl.ds(base, half)], iu0, sem_idx),
            pltpu.async_copy(ui_hbm.at[pl.ds(base + half, half)], iu1, sem_idx),
            pltpu.async_copy(mi_hbm.at[pl.ds(base, half)], im0, sem_idx),
            pltpu.async_copy(mi_hbm.at[pl.ds(base + half, half)], im1, sem_idx),
        ]
        for cp in idx_cps:
            cp.wait()
        items = [
            (ue_hbm, iu0, ou_hbm, 0),
            (ue_hbm, iu1, ou_hbm, half),
            (me_hbm, im0, om_hbm, 0),
            (me_hbm, im1, om_hbm, half),
        ]
        bufs = (buf0, buf1)
        sems = (sem0, sem1)
        pending = [None] * 4
        t0, i0, _, _ = items[0]
        pending[0] = pltpu.async_copy(t0.at[i0], bufs[0], sems[0])
        for i in range(4):
            if i + 1 < 4:
                t, idx, _, _ = items[i + 1]
                pending[i + 1] = pltpu.async_copy(
                    t.at[idx], bufs[(i + 1) % 2], sems[(i + 1) % 2])
            pending[i].wait()
            _, _, out, off = items[i]
            pltpu.sync_copy(bufs[i % 2], out.at[pl.ds(base + off, half)])

    return gather_kernel(user_emb, movie_emb, uids, mids)


_MLP_BLOCK = 4096


def _mlp_body(u_ref, m_ref, w1_ref, b1_ref, w2_ref, b2_ref,
              w3_ref, b3_ref, o_ref):
    D = u_ref.shape[1]
    h = jax.lax.dot_general(u_ref[...], w1_ref[:, :D], (((1,), (1,)), ((), ())),
                            preferred_element_type=jnp.float32)
    h = h + jax.lax.dot_general(m_ref[...], w1_ref[:, D:], (((1,), (1,)), ((), ())),
                                preferred_element_type=jnp.float32)
    h = jnp.maximum(h + b1_ref[...], 0.0)
    h2t = jax.lax.dot_general(w2_ref[...], h, (((1,), (1,)), ((), ())),
                              preferred_element_type=jnp.float32)
    h2t = jnp.maximum(h2t + b2_ref[...], 0.0)
    ot = jax.lax.dot_general(w3_ref[...], h2t, (((1,), (0,)), ((), ())),
                             preferred_element_type=jnp.float32)
    o_ref[...] = ot + b3_ref[0, 0]


def _mlp(u, m, W1, b1, W2, b2, W3, b3):
    B, D = u.shape
    b1_2d = b1.reshape(1, -1)
    b2_2d = b2.reshape(-1, 1)
    b3_2d = b3.reshape(1, 1)

    grid = (B // _MLP_BLOCK,)
    out_t = pl.pallas_call(
        _mlp_body,
        grid=grid,
        in_specs=[
            pl.BlockSpec((_MLP_BLOCK, D), lambda i: (i, 0)),
            pl.BlockSpec((_MLP_BLOCK, D), lambda i: (i, 0)),
            pl.BlockSpec(W1.shape, lambda i: (0, 0)),
            pl.BlockSpec(b1_2d.shape, lambda i: (0, 0)),
            pl.BlockSpec(W2.shape, lambda i: (0, 0)),
            pl.BlockSpec(b2_2d.shape, lambda i: (0, 0)),
            pl.BlockSpec(W3.shape, lambda i: (0, 0)),
            pl.BlockSpec(b3_2d.shape, lambda i: (0, 0)),
        ],
        out_specs=pl.BlockSpec((1, _MLP_BLOCK), lambda i: (0, i)),
        out_shape=jax.ShapeDtypeStruct((1, B), jnp.float32),
        compiler_params=pltpu.CompilerParams(
            dimension_semantics=("parallel",),
        ),
    )(u, m, W1, b1_2d, W2, b2_2d, W3, b3_2d)
    return out_t.reshape(B)


def kernel(user_ids, movie_ids, user_emb, movie_emb, W1, b1, W2, b2, W3, b3):
    u, m = _sc_gather(user_emb, movie_emb, user_ids, movie_ids)
    return _mlp(u, m, W1, b1, W2, b2, W3, b3)
